# baseline (speedup 1.0000x reference)
import jax
import jax.numpy as jnp
from jax.experimental import pallas as pl

BLK = 8192


def _copy_body(f_ref, idx_ref, fb_ref, idxb_ref, out_ref, idxo_ref, *, nf):
    i = pl.program_id(0)

    @pl.when(i < nf)
    def _():
        out_ref[...] = f_ref[...]
        idxo_ref[...] = idx_ref[...]

    @pl.when(i >= nf)
    def _():
        out_ref[...] = fb_ref[...]
        idxo_ref[...] = idxb_ref[...]


def kernel(f, idx, fb, idx_bank):
    f2 = f.reshape(-1, f.shape[-1])
    idx2 = idx.reshape(-1)
    N, F = f2.shape
    S = fb.shape[0]
    assert N % BLK == 0
    nf = N // BLK
    grid = (pl.cdiv(S, BLK),)

    import functools

    body = functools.partial(_copy_body, nf=nf)

    out_fb, out_idx = pl.pallas_call(
        body,
        grid=grid,
        in_specs=[
            pl.BlockSpec((BLK, F), lambda i: (jnp.minimum(i, nf - 1), 0)),
            pl.BlockSpec((BLK,), lambda i: (jnp.minimum(i, nf - 1),)),
            pl.BlockSpec((BLK, F), lambda i: (jnp.maximum(i - nf, 0), 0)),
            pl.BlockSpec((BLK,), lambda i: (jnp.maximum(i - nf, 0),)),
        ],
        out_specs=[
            pl.BlockSpec((BLK, F), lambda i: (i, 0)),
            pl.BlockSpec((BLK,), lambda i: (i,)),
        ],
        out_shape=[
            jax.ShapeDtypeStruct((S, F), fb.dtype),
            jax.ShapeDtypeStruct((S,), idx_bank.dtype),
        ],
    )(f2, idx2, fb, idx_bank)

    return (out_fb, out_idx)
